# baseline (speedup 1.0000x reference)
import functools

import jax
import jax.numpy as jnp
from jax.experimental import pallas as pl
from jax.experimental.pallas import tpu as pltpu


def _kmeans_body(x_ref, c_ref, centers_out_ref, counts_out_ref, c2_scr, *,
                 num_blocks, num_clusters, dim, bn):
    i = pl.program_id(0)

    @pl.when(i == 0)
    def _init():
        cc = c_ref[...]
        c2 = jnp.sum(cc * cc, axis=1, keepdims=True)
        c2_scr[...] = jnp.broadcast_to(c2, (num_clusters, 8))
        centers_out_ref[...] = jnp.zeros_like(centers_out_ref)
        counts_out_ref[...] = jnp.zeros_like(counts_out_ref)

    x = x_ref[...]
    scores = jax.lax.dot_general(
        c_ref[...], x, (((1,), (1,)), ((), ())),
        preferred_element_type=jnp.float32)
    val = scores - 0.5 * c2_scr[:, 0:1]
    mx = jnp.max(val, axis=0, keepdims=True)
    onehot = (val == mx).astype(jnp.bfloat16)
    centers_out_ref[...] += jax.lax.dot_general(
        onehot, x.astype(jnp.bfloat16), (((1,), (0,)), ((), ())),
        preferred_element_type=jnp.float32)
    counts_out_ref[...] += jax.lax.dot_general(
        onehot, jnp.ones((bn, 8), jnp.bfloat16), (((1,), (0,)), ((), ())),
        preferred_element_type=jnp.float32)

    @pl.when(i == num_blocks - 1)
    def _finalize():
        counts = counts_out_ref[:, 0:1]
        sums = centers_out_ref[...]
        means = sums / jnp.maximum(counts, 1.0)
        centers_out_ref[...] = jnp.where(counts > 0.0, means, c_ref[...])


@jax.jit
def kernel(x, cluster_centers):
    n, dim = x.shape
    num_clusters = cluster_centers.shape[0]
    bn = 8192
    num_blocks = n // bn

    new_centers, counts8 = pl.pallas_call(
        functools.partial(_kmeans_body, num_blocks=num_blocks,
                          num_clusters=num_clusters, dim=dim, bn=bn),
        grid=(num_blocks,),
        in_specs=[
            pl.BlockSpec((bn, dim), lambda i: (i, 0)),
            pl.BlockSpec((num_clusters, dim), lambda i: (0, 0)),
        ],
        out_specs=[
            pl.BlockSpec((num_clusters, dim), lambda i: (0, 0)),
            pl.BlockSpec((num_clusters, 8), lambda i: (0, 0)),
        ],
        out_shape=[
            jax.ShapeDtypeStruct((num_clusters, dim), jnp.float32),
            jax.ShapeDtypeStruct((num_clusters, 8), jnp.float32),
        ],
        scratch_shapes=[pltpu.VMEM((num_clusters, 8), jnp.float32)],
        compiler_params=pltpu.CompilerParams(
            dimension_semantics=("arbitrary",)),
    )(x, cluster_centers)

    return new_centers, counts8[:, 0]


# speedup vs baseline: 1.0092x; 1.0092x over previous
import functools

import jax
import jax.numpy as jnp
from jax.experimental import pallas as pl
from jax.experimental.pallas import tpu as pltpu


def _kmeans_body(x_ref, c_ref, centers_out_ref, counts_out_ref, c2_scr, *,
                 num_blocks, num_clusters, dim, bn):
    i = pl.program_id(0)

    @pl.when(i == 0)
    def _init():
        cc = c_ref[...]
        c2 = jnp.sum(cc * cc, axis=1, keepdims=True)
        c2_scr[...] = jnp.broadcast_to(c2, (num_clusters, 8))
        centers_out_ref[...] = jnp.zeros_like(centers_out_ref)
        counts_out_ref[...] = jnp.zeros_like(counts_out_ref)

    x = x_ref[...]
    scores = jax.lax.dot_general(
        c_ref[...], x, (((1,), (1,)), ((), ())),
        preferred_element_type=jnp.float32)
    val = scores - 0.5 * c2_scr[:, 0:1]
    mx = jnp.max(val, axis=0, keepdims=True)
    onehot = (val == mx).astype(jnp.float32)
    centers_out_ref[...] += jax.lax.dot_general(
        onehot, x, (((1,), (0,)), ((), ())),
        preferred_element_type=jnp.float32)
    counts_out_ref[...] += jax.lax.dot_general(
        onehot, jnp.ones((bn, 8), jnp.float32), (((1,), (0,)), ((), ())),
        preferred_element_type=jnp.float32)

    @pl.when(i == num_blocks - 1)
    def _finalize():
        counts = counts_out_ref[:, 0:1]
        sums = centers_out_ref[...]
        means = sums / jnp.maximum(counts, 1.0)
        centers_out_ref[...] = jnp.where(counts > 0.0, means, c_ref[...])


@jax.jit
def kernel(x, cluster_centers):
    n, dim = x.shape
    num_clusters = cluster_centers.shape[0]
    bn = 8192
    num_blocks = n // bn

    new_centers, counts8 = pl.pallas_call(
        functools.partial(_kmeans_body, num_blocks=num_blocks,
                          num_clusters=num_clusters, dim=dim, bn=bn),
        grid=(num_blocks,),
        in_specs=[
            pl.BlockSpec((bn, dim), lambda i: (i, 0)),
            pl.BlockSpec((num_clusters, dim), lambda i: (0, 0)),
        ],
        out_specs=[
            pl.BlockSpec((num_clusters, dim), lambda i: (0, 0)),
            pl.BlockSpec((num_clusters, 8), lambda i: (0, 0)),
        ],
        out_shape=[
            jax.ShapeDtypeStruct((num_clusters, dim), jnp.float32),
            jax.ShapeDtypeStruct((num_clusters, 8), jnp.float32),
        ],
        scratch_shapes=[pltpu.VMEM((num_clusters, 8), jnp.float32)],
        compiler_params=pltpu.CompilerParams(
            dimension_semantics=("arbitrary",)),
    )(x, cluster_centers)

    return new_centers, counts8[:, 0]


# speedup vs baseline: 1.1492x; 1.1387x over previous
import functools

import jax
import jax.numpy as jnp
from jax.experimental import pallas as pl
from jax.experimental.pallas import tpu as pltpu


def _kmeans_body(x_ref, c_ref, centers_out_ref, counts_out_ref, c2_scr, *,
                 num_blocks, num_clusters, dim, bn):
    i = pl.program_id(0)

    @pl.when(i == 0)
    def _init():
        cc = c_ref[...]
        c2 = jnp.sum(cc * cc, axis=1, keepdims=True)
        c2_scr[...] = jnp.broadcast_to(c2, (num_clusters, 8))
        centers_out_ref[...] = jnp.zeros_like(centers_out_ref)
        counts_out_ref[...] = jnp.zeros_like(counts_out_ref)

    x = x_ref[...]
    scores = jax.lax.dot_general(
        c_ref[...], x, (((1,), (1,)), ((), ())),
        preferred_element_type=jnp.float32)
    val = scores - 0.5 * c2_scr[:, 0:1]
    mx = jnp.max(val, axis=0, keepdims=True)
    onehot = (val == mx).astype(jnp.float32)
    centers_out_ref[...] += jax.lax.dot_general(
        onehot, x, (((1,), (0,)), ((), ())),
        preferred_element_type=jnp.float32)
    cnt = jnp.sum(onehot, axis=1, keepdims=True)
    counts_out_ref[...] += jnp.broadcast_to(cnt, (num_clusters, 8))

    @pl.when(i == num_blocks - 1)
    def _finalize():
        counts = counts_out_ref[:, 0:1]
        sums = centers_out_ref[...]
        means = sums / jnp.maximum(counts, 1.0)
        centers_out_ref[...] = jnp.where(counts > 0.0, means, c_ref[...])


@jax.jit
def kernel(x, cluster_centers):
    n, dim = x.shape
    num_clusters = cluster_centers.shape[0]
    bn = 8192
    num_blocks = n // bn

    new_centers, counts8 = pl.pallas_call(
        functools.partial(_kmeans_body, num_blocks=num_blocks,
                          num_clusters=num_clusters, dim=dim, bn=bn),
        grid=(num_blocks,),
        in_specs=[
            pl.BlockSpec((bn, dim), lambda i: (i, 0)),
            pl.BlockSpec((num_clusters, dim), lambda i: (0, 0)),
        ],
        out_specs=[
            pl.BlockSpec((num_clusters, dim), lambda i: (0, 0)),
            pl.BlockSpec((num_clusters, 8), lambda i: (0, 0)),
        ],
        out_shape=[
            jax.ShapeDtypeStruct((num_clusters, dim), jnp.float32),
            jax.ShapeDtypeStruct((num_clusters, 8), jnp.float32),
        ],
        scratch_shapes=[pltpu.VMEM((num_clusters, 8), jnp.float32)],
        compiler_params=pltpu.CompilerParams(
            dimension_semantics=("arbitrary",)),
    )(x, cluster_centers)

    return new_centers, counts8[:, 0]
